# baseline (speedup 1.0000x reference)
import functools

import jax
import jax.numpy as jnp
from jax import lax
from jax.experimental import pallas as pl
from jax.experimental.pallas import tpu as pltpu
from jax.experimental.pallas import tpu_sc as plsc

_ROWS_BLK = 1152


def _argmin_body(z_ref, cb_ref, idx_ref):
    zb = z_ref[...]
    cb = cb_ref[...]
    dot = lax.dot_general(
        zb, cb, (((1,), (1,)), ((), ())),
        preferred_element_type=jnp.float32,
    )
    zsq = jnp.sum(zb * zb, axis=1, keepdims=True)
    csq = jnp.sum(cb * cb, axis=1)[None, :]
    d = zsq + csq - 2.0 * dot
    mins = jnp.min(d, axis=1, keepdims=True)
    ids = lax.broadcasted_iota(jnp.int32, d.shape, 1)
    first = jnp.min(jnp.where(d == mins, ids, jnp.int32(2 ** 30)), axis=1)
    idx_ref[0, 0, :] = first


def _codebook_argmin(zf, codebook):
    n, d = zf.shape
    n_e = codebook.shape[0]
    nblk = n // _ROWS_BLK
    out = pl.pallas_call(
        _argmin_body,
        grid=(nblk,),
        in_specs=[
            pl.BlockSpec((_ROWS_BLK, d), lambda i: (i, 0)),
            pl.BlockSpec((n_e, d), lambda i: (0, 0)),
        ],
        out_specs=pl.BlockSpec((1, 1, _ROWS_BLK), lambda i: (i, 0, 0)),
        out_shape=jax.ShapeDtypeStruct((nblk, 1, _ROWS_BLK), jnp.int32),
    )(zf, codebook)
    return out.reshape(n)


def _sc_gather(codebook, idx):
    b = idx.shape[0]
    d = codebook.shape[1]
    info = plsc.get_sparse_core_info()
    nc, ns = info.num_cores, info.num_subcores
    nw = nc * ns
    b_per_w = b // nw
    nch = 1
    while b_per_w // nch > 128 or (b_per_w // nch) % 8:
        nch += 1
    ch = b_per_w // nch
    mesh = plsc.VectorSubcoreMesh(core_axis_name="c", subcore_axis_name="s")

    @functools.partial(
        pl.kernel,
        out_type=jax.ShapeDtypeStruct((b, d), jnp.float32),
        mesh=mesh,
        scratch_types=[
            pltpu.VMEM((ch,), jnp.int32),
            pltpu.VMEM((ch, d), jnp.float32),
            pltpu.SemaphoreType.DMA,
        ],
        compiler_params=pltpu.CompilerParams(use_tc_tiling_on_sc=False),
    )
    def gather_k(table_hbm, idx_hbm, out_hbm, idx_v, rows_v, sem):
        wid = lax.axis_index("s") * nc + lax.axis_index("c")
        base = wid * b_per_w
        for j in range(nch):
            off = base + j * ch
            pltpu.sync_copy(idx_hbm.at[pl.ds(off, ch)], idx_v)
            pltpu.async_copy(table_hbm.at[idx_v], rows_v, sem).wait()
            pltpu.sync_copy(rows_v, out_hbm.at[pl.ds(off, ch)])

    return gather_k(codebook, idx)


_ROT_BLK = 1152


def _rot_body(n_total, zf_ref, zq_ref, out_ref, loss_ref, acc_ref):
    eps = 1e-6
    zf = zf_ref[...]
    zq = zq_ref[...]
    n_src = jnp.sqrt(jnp.sum(zf * zf, axis=1, keepdims=True))
    n_tgt = jnp.sqrt(jnp.sum(zq * zq, axis=1, keepdims=True))
    u = zf / jnp.maximum(n_src, eps)
    q = zq / jnp.maximum(n_tgt, eps)
    wv = u + q
    wn = jnp.sqrt(jnp.sum(wv * wv, axis=1, keepdims=True))
    w = wv / jnp.maximum(wn, eps)
    ew = jnp.sum(zf * w, axis=1, keepdims=True)
    eu = jnp.sum(zf * u, axis=1, keepdims=True)
    rot = zf - 2.0 * ew * w + 2.0 * eu * q
    out_ref[...] = rot * (n_tgt / jnp.maximum(n_src, eps))
    diff = zq - zf

    @pl.when(pl.program_id(0) == 0)
    def _init():
        acc_ref[0] = 0.0

    acc_ref[0] += jnp.sum(diff * diff)

    @pl.when(pl.program_id(0) == pl.num_programs(0) - 1)
    def _fin():
        loss_ref[0, 0] = 2.0 * acc_ref[0] / n_total


def _rotate_and_loss(zf, z_q, n_total):
    n, d = zf.shape
    nblk = n // _ROT_BLK
    return pl.pallas_call(
        functools.partial(_rot_body, float(n_total)),
        grid=(nblk,),
        in_specs=[
            pl.BlockSpec((_ROT_BLK, d), lambda i: (i, 0)),
            pl.BlockSpec((_ROT_BLK, d), lambda i: (i, 0)),
        ],
        out_specs=[
            pl.BlockSpec((_ROT_BLK, d), lambda i: (i, 0)),
            pl.BlockSpec((1, 1), lambda i: (0, 0), memory_space=pltpu.SMEM),
        ],
        out_shape=[
            jax.ShapeDtypeStruct(zf.shape, jnp.float32),
            jax.ShapeDtypeStruct((1, 1), jnp.float32),
        ],
        scratch_shapes=[pltpu.SMEM((1,), jnp.float32)],
    )(zf, z_q)


def kernel(z, codebook):
    zf = z.reshape(-1, z.shape[-1])
    n, d = zf.shape
    h = n // 2
    zfa, zfb = zf[:h], zf[h:]
    idx_a = _codebook_argmin(zfa, codebook)
    zq_a = _sc_gather(codebook, idx_a)
    idx_b = _codebook_argmin(zfb, codebook)
    zq_b = _sc_gather(codebook, idx_b)
    out_a, loss_a = _rotate_and_loss(zfa, zq_a, n * d)
    out_b, loss_b = _rotate_and_loss(zfb, zq_b, n * d)
    out = jnp.concatenate([out_a, out_b], axis=0)
    return out.reshape(z.shape), loss_a[0, 0] + loss_b[0, 0]


# speedup vs baseline: 1.0600x; 1.0600x over previous
import functools

import jax
import jax.numpy as jnp
from jax import lax
from jax.experimental import pallas as pl
from jax.experimental.pallas import tpu as pltpu
from jax.experimental.pallas import tpu_sc as plsc

_ROWS_BLK = 1152


def _argmin_body(z_ref, cb_ref, idx_ref):
    zb = z_ref[...]
    cb = cb_ref[...]
    dot = lax.dot_general(
        zb, cb, (((1,), (1,)), ((), ())),
        preferred_element_type=jnp.float32,
    )
    zsq = jnp.sum(zb * zb, axis=1, keepdims=True)
    csq = jnp.sum(cb * cb, axis=1)[None, :]
    d = zsq + csq - 2.0 * dot
    mins = jnp.min(d, axis=1, keepdims=True)
    ids = lax.broadcasted_iota(jnp.int32, (1, d.shape[1]), 1)
    first = jnp.min(jnp.where(d == mins, ids, jnp.int32(2 ** 30)), axis=1)
    idx_ref[0, 0, :] = first


def _codebook_argmin(zf, codebook):
    n, d = zf.shape
    n_e = codebook.shape[0]
    nblk = n // _ROWS_BLK
    out = pl.pallas_call(
        _argmin_body,
        grid=(nblk,),
        in_specs=[
            pl.BlockSpec((_ROWS_BLK, d), lambda i: (i, 0)),
            pl.BlockSpec((n_e, d), lambda i: (0, 0)),
        ],
        out_specs=pl.BlockSpec((1, 1, _ROWS_BLK), lambda i: (i, 0, 0)),
        out_shape=jax.ShapeDtypeStruct((nblk, 1, _ROWS_BLK), jnp.int32),
    )(zf, codebook)
    return out.reshape(n)


def _sc_gather(codebook, idx):
    b = idx.shape[0]
    d = codebook.shape[1]
    info = plsc.get_sparse_core_info()
    nc, ns = info.num_cores, info.num_subcores
    nw = nc * ns
    b_per_w = b // nw
    nch = 1
    while b_per_w // nch > 128 or (b_per_w // nch) % 8:
        nch += 1
    ch = b_per_w // nch
    mesh = plsc.VectorSubcoreMesh(core_axis_name="c", subcore_axis_name="s")

    @functools.partial(
        pl.kernel,
        out_type=jax.ShapeDtypeStruct((b, d), jnp.float32),
        mesh=mesh,
        scratch_types=[
            pltpu.VMEM((ch,), jnp.int32),
            pltpu.VMEM((ch, d), jnp.float32),
            pltpu.SemaphoreType.DMA,
        ],
        compiler_params=pltpu.CompilerParams(use_tc_tiling_on_sc=False),
    )
    def gather_k(table_hbm, idx_hbm, out_hbm, idx_v, rows_v, sem):
        wid = lax.axis_index("s") * nc + lax.axis_index("c")
        base = wid * b_per_w
        for j in range(nch):
            off = base + j * ch
            pltpu.sync_copy(idx_hbm.at[pl.ds(off, ch)], idx_v)
            pltpu.async_copy(table_hbm.at[idx_v], rows_v, sem).wait()
            pltpu.sync_copy(rows_v, out_hbm.at[pl.ds(off, ch)])

    return gather_k(codebook, idx)


_ROT_BLK = 1152


def _rot_body(n_total, zf_ref, zq_ref, out_ref, loss_ref, acc_ref):
    eps = 1e-6
    zf = zf_ref[...]
    zq = zq_ref[...]
    n_src = jnp.sqrt(jnp.sum(zf * zf, axis=1, keepdims=True))
    n_tgt = jnp.sqrt(jnp.sum(zq * zq, axis=1, keepdims=True))
    u = zf / jnp.maximum(n_src, eps)
    q = zq / jnp.maximum(n_tgt, eps)
    wv = u + q
    wn = jnp.sqrt(jnp.sum(wv * wv, axis=1, keepdims=True))
    w = wv / jnp.maximum(wn, eps)
    ew = jnp.sum(zf * w, axis=1, keepdims=True)
    eu = jnp.sum(zf * u, axis=1, keepdims=True)
    rot = zf - 2.0 * ew * w + 2.0 * eu * q
    out_ref[...] = rot * (n_tgt / jnp.maximum(n_src, eps))
    diff = zq - zf

    @pl.when(pl.program_id(0) == 0)
    def _init():
        acc_ref[0] = 0.0

    acc_ref[0] += jnp.sum(diff * diff)

    @pl.when(pl.program_id(0) == pl.num_programs(0) - 1)
    def _fin():
        loss_ref[0, 0] = 2.0 * acc_ref[0] / n_total


def _rotate_and_loss(zf, z_q, n_total):
    n, d = zf.shape
    nblk = n // _ROT_BLK
    return pl.pallas_call(
        functools.partial(_rot_body, float(n_total)),
        grid=(nblk,),
        in_specs=[
            pl.BlockSpec((_ROT_BLK, d), lambda i: (i, 0)),
            pl.BlockSpec((_ROT_BLK, d), lambda i: (i, 0)),
        ],
        out_specs=[
            pl.BlockSpec((_ROT_BLK, d), lambda i: (i, 0)),
            pl.BlockSpec((1, 1), lambda i: (0, 0), memory_space=pltpu.SMEM),
        ],
        out_shape=[
            jax.ShapeDtypeStruct(zf.shape, jnp.float32),
            jax.ShapeDtypeStruct((1, 1), jnp.float32),
        ],
        scratch_shapes=[pltpu.SMEM((1,), jnp.float32)],
    )(zf, z_q)


def kernel(z, codebook):
    zf = z.reshape(-1, z.shape[-1])
    n, d = zf.shape
    idx = _codebook_argmin(zf, codebook)
    z_q = _sc_gather(codebook, idx)
    out, loss = _rotate_and_loss(zf, z_q, n * d)
    return out.reshape(z.shape), loss[0, 0]


# speedup vs baseline: 1.0677x; 1.0072x over previous
import functools

import jax
import jax.numpy as jnp
from jax import lax
from jax.experimental import pallas as pl
from jax.experimental.pallas import tpu as pltpu
from jax.experimental.pallas import tpu_sc as plsc

_ROWS_BLK = 1152


def _argmin_body(z_ref, cb_ref, idx_ref):
    zb = z_ref[...]
    cb = cb_ref[...]
    dot = lax.dot_general(
        zb, cb, (((1,), (1,)), ((), ())),
        preferred_element_type=jnp.float32,
    )
    zsq = jnp.sum(zb * zb, axis=1, keepdims=True)
    csq = jnp.sum(cb * cb, axis=1)[None, :]
    d = zsq + csq - 2.0 * dot
    mins = jnp.min(d, axis=1, keepdims=True)
    ids = lax.broadcasted_iota(jnp.int32, (1, d.shape[1]), 1)
    first = jnp.min(jnp.where(d == mins, ids, jnp.int32(2 ** 30)), axis=1)
    idx_ref[0, 0, :] = first


def _codebook_argmin(zf, codebook):
    n, d = zf.shape
    n_e = codebook.shape[0]
    nblk = n // _ROWS_BLK
    out = pl.pallas_call(
        _argmin_body,
        grid=(nblk,),
        in_specs=[
            pl.BlockSpec((_ROWS_BLK, d), lambda i: (i, 0)),
            pl.BlockSpec((n_e, d), lambda i: (0, 0)),
        ],
        out_specs=pl.BlockSpec((1, 1, _ROWS_BLK), lambda i: (i, 0, 0)),
        out_shape=jax.ShapeDtypeStruct((nblk, 1, _ROWS_BLK), jnp.int32),
    )(zf, codebook)
    return out.reshape(n)


def _sc_gather(codebook, idx):
    b = idx.shape[0]
    d = codebook.shape[1]
    info = plsc.get_sparse_core_info()
    nc, ns = info.num_cores, info.num_subcores
    nw = nc * ns
    b_per_w = b // nw
    nch = 1
    while b_per_w // nch > 128 or (b_per_w // nch) % 8:
        nch += 1
    ch = b_per_w // nch
    mesh = plsc.VectorSubcoreMesh(core_axis_name="c", subcore_axis_name="s")

    @functools.partial(
        pl.kernel,
        out_type=jax.ShapeDtypeStruct((b, d), jnp.float32),
        mesh=mesh,
        scratch_types=[
            pltpu.VMEM((nch, ch), jnp.int32),
            pltpu.VMEM((nch, ch, d), jnp.float32),
            pltpu.SemaphoreType.DMA,
        ],
        compiler_params=pltpu.CompilerParams(use_tc_tiling_on_sc=False),
    )
    def gather_k(table_hbm, idx_hbm, out_hbm, idx_v, rows_v, sem):
        wid = lax.axis_index("s") * nc + lax.axis_index("c")
        base = wid * b_per_w
        for j in range(nch):
            pltpu.sync_copy(idx_hbm.at[pl.ds(base + j * ch, ch)], idx_v.at[j])
        gathers = [
            pltpu.async_copy(table_hbm.at[idx_v.at[j]], rows_v.at[j], sem)
            for j in range(nch)
        ]
        for j in range(nch):
            gathers[j].wait()
            pltpu.sync_copy(rows_v.at[j], out_hbm.at[pl.ds(base + j * ch, ch)])

    return gather_k(codebook, idx)


_ROT_BLK = 1152


def _rot_body(n_total, zf_ref, zq_ref, out_ref, loss_ref, acc_ref):
    eps = 1e-6
    zf = zf_ref[...]
    zq = zq_ref[...]
    n_src = jnp.sqrt(jnp.sum(zf * zf, axis=1, keepdims=True))
    n_tgt = jnp.sqrt(jnp.sum(zq * zq, axis=1, keepdims=True))
    u = zf / jnp.maximum(n_src, eps)
    q = zq / jnp.maximum(n_tgt, eps)
    wv = u + q
    wn = jnp.sqrt(jnp.sum(wv * wv, axis=1, keepdims=True))
    w = wv / jnp.maximum(wn, eps)
    ew = jnp.sum(zf * w, axis=1, keepdims=True)
    eu = jnp.sum(zf * u, axis=1, keepdims=True)
    rot = zf - 2.0 * ew * w + 2.0 * eu * q
    out_ref[...] = rot * (n_tgt / jnp.maximum(n_src, eps))
    diff = zq - zf

    @pl.when(pl.program_id(0) == 0)
    def _init():
        acc_ref[0] = 0.0

    acc_ref[0] += jnp.sum(diff * diff)

    @pl.when(pl.program_id(0) == pl.num_programs(0) - 1)
    def _fin():
        loss_ref[0, 0] = 2.0 * acc_ref[0] / n_total


def _rotate_and_loss(zf, z_q, n_total):
    n, d = zf.shape
    nblk = n // _ROT_BLK
    return pl.pallas_call(
        functools.partial(_rot_body, float(n_total)),
        grid=(nblk,),
        in_specs=[
            pl.BlockSpec((_ROT_BLK, d), lambda i: (i, 0)),
            pl.BlockSpec((_ROT_BLK, d), lambda i: (i, 0)),
        ],
        out_specs=[
            pl.BlockSpec((_ROT_BLK, d), lambda i: (i, 0)),
            pl.BlockSpec((1, 1), lambda i: (0, 0), memory_space=pltpu.SMEM),
        ],
        out_shape=[
            jax.ShapeDtypeStruct(zf.shape, jnp.float32),
            jax.ShapeDtypeStruct((1, 1), jnp.float32),
        ],
        scratch_shapes=[pltpu.SMEM((1,), jnp.float32)],
    )(zf, z_q)


def kernel(z, codebook):
    zf = z.reshape(-1, z.shape[-1])
    n, d = zf.shape
    idx = _codebook_argmin(zf, codebook)
    z_q = _sc_gather(codebook, idx)
    out, loss = _rotate_and_loss(zf, z_q, n * d)
    return out.reshape(z.shape), loss[0, 0]


# speedup vs baseline: 1.1659x; 1.0920x over previous
import functools

import jax
import jax.numpy as jnp
from jax import lax
from jax.experimental import pallas as pl
from jax.experimental.pallas import tpu as pltpu
from jax.experimental.pallas import tpu_sc as plsc

_ROWS_BLK = 1152


def _argmin_body(z_ref, cb_ref, idx_ref):
    zb = z_ref[...]
    cb = cb_ref[...]
    dot = lax.dot_general(
        zb, cb, (((1,), (1,)), ((), ())),
        preferred_element_type=jnp.float32,
    )
    zsq = jnp.sum(zb * zb, axis=1, keepdims=True)
    csq = jnp.sum(cb * cb, axis=1)[None, :]
    d = zsq + csq - 2.0 * dot
    mins = jnp.min(d, axis=1, keepdims=True)
    ids = lax.broadcasted_iota(jnp.int32, (1, d.shape[1]), 1).astype(jnp.float32)
    first = jnp.min(jnp.where(d == mins, ids, jnp.float32(2 ** 24)), axis=1)
    idx_ref[0, 0, :] = first.astype(jnp.int32)


def _codebook_argmin(zf, codebook):
    n, d = zf.shape
    n_e = codebook.shape[0]
    nblk = n // _ROWS_BLK
    out = pl.pallas_call(
        _argmin_body,
        grid=(nblk,),
        in_specs=[
            pl.BlockSpec((_ROWS_BLK, d), lambda i: (i, 0)),
            pl.BlockSpec((n_e, d), lambda i: (0, 0)),
        ],
        out_specs=pl.BlockSpec((1, 1, _ROWS_BLK), lambda i: (i, 0, 0)),
        out_shape=jax.ShapeDtypeStruct((nblk, 1, _ROWS_BLK), jnp.int32),
    )(zf, codebook)
    return out.reshape(n)


def _sc_gather(codebook, idx):
    b = idx.shape[0]
    d = codebook.shape[1]
    info = plsc.get_sparse_core_info()
    nc, ns = info.num_cores, info.num_subcores
    nw = nc * ns
    b_per_w = b // nw
    nch = 1
    while b_per_w // nch > 128 or (b_per_w // nch) % 8:
        nch += 1
    ch = b_per_w // nch
    mesh = plsc.VectorSubcoreMesh(core_axis_name="c", subcore_axis_name="s")

    @functools.partial(
        pl.kernel,
        out_type=jax.ShapeDtypeStruct((b, d), jnp.float32),
        mesh=mesh,
        scratch_types=[
            pltpu.VMEM((nch, ch), jnp.int32),
            pltpu.VMEM((nch, ch, d), jnp.float32),
            pltpu.SemaphoreType.DMA,
        ],
        compiler_params=pltpu.CompilerParams(use_tc_tiling_on_sc=False),
    )
    def gather_k(table_hbm, idx_hbm, out_hbm, idx_v, rows_v, sem):
        wid = lax.axis_index("s") * nc + lax.axis_index("c")
        base = wid * b_per_w
        for j in range(nch):
            pltpu.sync_copy(idx_hbm.at[pl.ds(base + j * ch, ch)], idx_v.at[j])
        gathers = [
            pltpu.async_copy(table_hbm.at[idx_v.at[j]], rows_v.at[j], sem)
            for j in range(nch)
        ]
        for j in range(nch):
            gathers[j].wait()
            pltpu.sync_copy(rows_v.at[j], out_hbm.at[pl.ds(base + j * ch, ch)])

    return gather_k(codebook, idx)


_ROT_BLK = 1152


def _rot_body(n_total, zf_ref, zq_ref, out_ref, loss_ref, acc_ref):
    eps = 1e-6
    zf = zf_ref[...]
    zq = zq_ref[...]
    n_src = jnp.sqrt(jnp.sum(zf * zf, axis=1, keepdims=True))
    n_tgt = jnp.sqrt(jnp.sum(zq * zq, axis=1, keepdims=True))
    u = zf / jnp.maximum(n_src, eps)
    q = zq / jnp.maximum(n_tgt, eps)
    wv = u + q
    wn = jnp.sqrt(jnp.sum(wv * wv, axis=1, keepdims=True))
    w = wv / jnp.maximum(wn, eps)
    ew = jnp.sum(zf * w, axis=1, keepdims=True)
    eu = jnp.sum(zf * u, axis=1, keepdims=True)
    rot = zf - 2.0 * ew * w + 2.0 * eu * q
    out_ref[...] = rot * (n_tgt / jnp.maximum(n_src, eps))
    diff = zq - zf

    @pl.when(pl.program_id(0) == 0)
    def _init():
        acc_ref[0] = 0.0

    acc_ref[0] += jnp.sum(diff * diff)

    @pl.when(pl.program_id(0) == pl.num_programs(0) - 1)
    def _fin():
        loss_ref[0, 0] = 2.0 * acc_ref[0] / n_total


def _rotate_and_loss(zf, z_q, n_total):
    n, d = zf.shape
    nblk = n // _ROT_BLK
    return pl.pallas_call(
        functools.partial(_rot_body, float(n_total)),
        grid=(nblk,),
        in_specs=[
            pl.BlockSpec((_ROT_BLK, d), lambda i: (i, 0)),
            pl.BlockSpec((_ROT_BLK, d), lambda i: (i, 0)),
        ],
        out_specs=[
            pl.BlockSpec((_ROT_BLK, d), lambda i: (i, 0)),
            pl.BlockSpec((1, 1), lambda i: (0, 0), memory_space=pltpu.SMEM),
        ],
        out_shape=[
            jax.ShapeDtypeStruct(zf.shape, jnp.float32),
            jax.ShapeDtypeStruct((1, 1), jnp.float32),
        ],
        scratch_shapes=[pltpu.SMEM((1,), jnp.float32)],
    )(zf, z_q)


def kernel(z, codebook):
    zf = z.reshape(-1, z.shape[-1])
    n, d = zf.shape
    idx = _codebook_argmin(zf, codebook)
    z_q = _sc_gather(codebook, idx)
    out, loss = _rotate_and_loss(zf, z_q, n * d)
    return out.reshape(z.shape), loss[0, 0]
